# baseline (speedup 1.0000x reference)
import functools

import jax
import jax.numpy as jnp
from jax import lax
from jax.experimental import pallas as pl
from jax.experimental.pallas import tpu as pltpu
from jax.experimental.pallas import tpu_sc as plsc

EMBED_DIM = 32
NUM_CORES = 2
NUM_SUBCORES = 16
NW = NUM_CORES * NUM_SUBCORES
NB = 8


@functools.lru_cache(maxsize=None)
def _make_kernel(batch: int, hist: int):
    per_w = batch // NW
    n_blocks = per_w // NB
    mesh = plsc.VectorSubcoreMesh(core_axis_name="c", subcore_axis_name="s")

    @functools.partial(
        pl.kernel,
        mesh=mesh,
        compiler_params=pltpu.CompilerParams(use_tc_tiling_on_sc=False),
        out_type=jax.ShapeDtypeStruct((batch, hist, EMBED_DIM), jnp.float32),
        scratch_types=[
            pltpu.VMEM((per_w, hist), jnp.int32),
            pltpu.VMEM((2, NB, hist, EMBED_DIM), jnp.float32),
            pltpu.SemaphoreType.DMA,
            pltpu.SemaphoreType.DMA,
        ],
    )
    def emb(x_hbm, table_hbm, out_hbm, idx_v, rows_v, gsem, wsem):
        wid = lax.axis_index("s") * NUM_CORES + lax.axis_index("c")
        base = wid * per_w
        pltpu.sync_copy(x_hbm.at[wid], idx_v)

        def gather(tb, s, b):
            return pltpu.make_async_copy(
                table_hbm.at[idx_v.at[tb * NB + b]],
                rows_v.at[s].at[b],
                gsem,
            )

        def write(tb, s):
            return pltpu.make_async_copy(
                rows_v.at[s], out_hbm.at[pl.ds(base + tb * NB, NB)], wsem)

        for b in range(NB):
            gather(0, 0, b).start()

        def body(tb, carry):
            s = lax.rem(tb, 2)
            for b in range(NB):
                gather(tb, s, b).wait()
            @pl.when(tb >= 1)
            def _():
                write(tb - 1, 1 - s).wait()
            write(tb, s).start()
            @pl.when(tb + 1 < n_blocks)
            def _():
                for b in range(NB):
                    gather(tb + 1, 1 - s, b).start()
            return carry

        lax.fori_loop(0, n_blocks, body, 0)
        write(n_blocks - 1, (n_blocks - 1) % 2).wait()

    return emb


def kernel(x, table):
    b, h = x.shape
    xr = x.astype(jnp.int32).reshape(NW, b // NW, h)
    return _make_kernel(b, h)(xr, table)


# speedup vs baseline: 1.3862x; 1.3862x over previous
import functools

import jax
import jax.numpy as jnp
from jax import lax
from jax.experimental import pallas as pl
from jax.experimental.pallas import tpu as pltpu
from jax.experimental.pallas import tpu_sc as plsc

EMBED_DIM = 32
HPAD = 56
EPAD = 128
NUM_CORES = 2
NUM_SUBCORES = 16
NW = NUM_CORES * NUM_SUBCORES
NB = 8


@functools.lru_cache(maxsize=None)
def _make_kernel(batch: int, hist: int):
    per_w = batch // NW
    n_blocks = per_w // NB
    mesh = plsc.VectorSubcoreMesh(core_axis_name="c", subcore_axis_name="s")

    @functools.partial(
        pl.kernel,
        mesh=mesh,
        compiler_params=pltpu.CompilerParams(use_tc_tiling_on_sc=False),
        out_type=jax.ShapeDtypeStruct((batch, HPAD, EPAD), jnp.float32),
        scratch_types=[
            pltpu.VMEM((per_w, hist), jnp.int32),
            pltpu.VMEM((2, NB, hist, EMBED_DIM), jnp.float32),
            pltpu.SemaphoreType.DMA,
            pltpu.SemaphoreType.DMA,
        ],
    )
    def emb(x_hbm, table_hbm, out_hbm, idx_v, rows_v, gsem, wsem):
        wid = lax.axis_index("s") * NUM_CORES + lax.axis_index("c")
        base = wid * per_w
        pltpu.sync_copy(x_hbm.at[wid], idx_v)

        def gather(tb, s, b):
            return pltpu.make_async_copy(
                table_hbm.at[idx_v.at[tb * NB + b]],
                rows_v.at[s].at[b],
                gsem,
            )

        def write(tb, s, b):
            return pltpu.make_async_copy(
                rows_v.at[s].at[b],
                out_hbm.at[base + tb * NB + b].at[pl.ds(0, hist),
                                                  pl.ds(0, EMBED_DIM)],
                wsem,
            )

        for b in range(NB):
            gather(0, 0, b).start()

        def body(tb, carry):
            s = lax.rem(tb, 2)
            for b in range(NB):
                gather(tb, s, b).wait()
            @pl.when(tb >= 1)
            def _():
                for b in range(NB):
                    write(tb - 1, 1 - s, b).wait()
            for b in range(NB):
                write(tb, s, b).start()
            @pl.when(tb + 1 < n_blocks)
            def _():
                for b in range(NB):
                    gather(tb + 1, 1 - s, b).start()
            return carry

        lax.fori_loop(0, n_blocks, body, 0)
        for b in range(NB):
            write(n_blocks - 1, lax.rem(n_blocks - 1, 2), b).wait()

    return emb


def kernel(x, table):
    b, h = x.shape
    xr = x.astype(jnp.int32).reshape(NW, b // NW, h)
    outp = _make_kernel(b, h)(xr, table)
    return outp[:, :h, :EMBED_DIM]
